# baseline (speedup 1.0000x reference)
import functools

import jax
import jax.numpy as jnp
from jax import lax
from jax.experimental import pallas as pl
from jax.experimental.pallas import tpu as pltpu
from jax.experimental.pallas import tpu_sc as plsc

N_USERS = 5000
N_ITEMS = 5000
E = 320000
D = 128
B = 4096
NUM_LAYERS = 3

NC = 2
NS = 16
L = 16

NPAD = 5120
ROWS_PER_SUB = NPAD // NS
C = 128
EP = 20480
E_PAD = EP * NS
NCHUNK = EP // C
NCHUNK_ALL = NCHUNK * NS

B_PER_W = B // (NC * NS)

_BCAST_DNUMS = lax.GatherDimensionNumbers(
    offset_dims=(), collapsed_slice_dims=(0,), start_index_map=(0,))


def _bcast_lane(vec16, j):
    idx = jnp.full((L, 1), j, jnp.int32)
    return lax.gather(vec16, idx, _BCAST_DNUMS, (1,),
                      mode=lax.GatherScatterMode.PROMISE_IN_BOUNDS)


def _scale_rows(rows_bf_ref, rows_f_ref, pk_ref):
    def body(g, carry):
        norms16 = plsc.bitcast(pk_ref[2, pl.ds(g * L, L)], jnp.float32)
        def ebody(j, carry2):
            e = g * L + j
            nb = _bcast_lane(norms16, j)
            for d in range(D // (2 * L)):
                vi = rows_bf_ref[e, pl.ds(d * L, L)]
                v = plsc.bitcast(vi, jnp.bfloat16)
                a, b2 = plsc.unpack(v, format=plsc.PackFormat.INTERLEAVED)
                rows_f_ref[e, pl.ds(d * 2 * L, L)] = a * nb
                rows_f_ref[e, pl.ds(d * 2 * L + L, L)] = b2 * nb
            return carry2
        lax.fori_loop(0, L, ebody, 0)
        return carry
    lax.fori_loop(0, C // L, body, 0)


NPK = 4
NRW = 2
CB = 32


def _layer_body(hu, hi, pk_ui, pk_iu, new_u, new_i,
                acc_sh, table_sh, pks, rowss, rows_bfs,
                isems, gsems, ssems):
    c = lax.axis_index("c")
    s = lax.axis_index("s")
    pk = tuple(pks)
    rows = tuple(rowss)
    rows_bf = tuple(rows_bfs)
    isem = tuple(isems)
    gsem = tuple(gsems)
    ssem = tuple(ssems)

    cpy = rowss[0].at[pl.ds(0, CB)]
    cbv = rows_bfs[0].at[pl.ds(0, CB)]

    z16 = jnp.zeros((L,), jnp.float32)
    def zbody(r, carry):
        for d in range(D // L):
            cpy[r, pl.ds(d * L, L)] = z16
        return carry
    lax.fori_loop(0, CB, zbody, 0)
    def zcp(t, carry):
        pltpu.sync_copy(cpy,
                        acc_sh.at[pl.ds(s * ROWS_PER_SUB + t * CB, CB)])
        return carry
    lax.fori_loop(0, ROWS_PER_SUB // CB, zcp, 0)
    plsc.subcore_barrier()

    def do_dir(table, packed, out):
        cbase = s * NCHUNK

        def tcp(t, carry):
            sl = pl.ds(s * ROWS_PER_SUB + t * CB, CB)
            pltpu.sync_copy(table.at[sl], cpy)
            def cvt(r, carry2):
                for d in range(D // (2 * L)):
                    a = cpy[r, pl.ds(d * 2 * L, L)]
                    b2 = cpy[r, pl.ds(d * 2 * L + L, L)]
                    pk_bf = plsc.pack(a, b2,
                                      format=plsc.PackFormat.INTERLEAVED)
                    cbv[r, pl.ds(d * L, L)] = plsc.bitcast(pk_bf, jnp.int32)
                return carry2
            lax.fori_loop(0, CB, cvt, 0)
            pltpu.sync_copy(cbv, table_sh.at[sl])
            return carry
        lax.fori_loop(0, ROWS_PER_SUB // CB, tcp, 0)
        plsc.subcore_barrier()

        def idx_start(kc, slot):
            pltpu.make_async_copy(packed.at[cbase + kc], pk[slot],
                                  isem[slot]).start()

        def idx_wait(kc, slot):
            pltpu.make_async_copy(packed.at[cbase + kc], pk[slot],
                                  isem[slot]).wait()

        def gat_start(pslot, rslot):
            pltpu.make_async_copy(table_sh.at[pk[pslot].at[0]],
                                  rows_bf[rslot], gsem[rslot]).start()

        def gat_wait(rslot):
            pltpu.make_async_copy(table_sh.at[pk[0].at[0]],
                                  rows_bf[rslot], gsem[rslot]).wait()

        def scat_start(pslot, rslot):
            pltpu.make_async_copy(rows[rslot], acc_sh.at[pk[pslot].at[1]],
                                  ssem[rslot]).start(add=True)

        def scat_wait(rslot):
            pltpu.make_async_copy(rows[rslot], acc_sh.at[pk[0].at[1]],
                                  ssem[rslot]).wait()

        idx_start(0, 0)
        idx_start(1, 1)
        idx_wait(0, 0)
        gat_start(0, 0)

        def body4(k4, carry):
            for b in range(NPK):
                k = k4 * NPK + b
                rb = b % NRW
                rn = (b + 1) % NRW
                pn1 = (b + 1) % NPK
                pn2 = (b + 2) % NPK

                @pl.when(k < NCHUNK - 1)
                def _():
                    idx_wait(k + 1, pn1)

                if b == 0:
                    @pl.when(k > 0)
                    def _():
                        scat_wait(rn)
                else:
                    scat_wait(rn)

                @pl.when(k < NCHUNK - 1)
                def _():
                    gat_start(pn1, rn)

                @pl.when(k < NCHUNK - 2)
                def _():
                    idx_start(k + 2, pn2)

                gat_wait(rb)
                _scale_rows(rows_bf[rb], rows[rb], pk[b])
                scat_start(b, rb)
            return carry

        lax.fori_loop(0, NCHUNK // NPK, body4, 0)
        scat_wait((NCHUNK - 1) % NRW)
        plsc.subcore_barrier()
        def ocp(t, carry):
            sl = pl.ds(s * ROWS_PER_SUB + t * CB, CB)
            pltpu.sync_copy(acc_sh.at[sl], cpy)
            pltpu.sync_copy(cpy, out.at[sl])
            return carry
        lax.fori_loop(0, ROWS_PER_SUB // CB, ocp, 0)

    @pl.when(c == 0)
    def _():
        do_dir(hu, pk_ui, new_i)

    @pl.when(c == 1)
    def _():
        do_dir(hi, pk_iu, new_u)


_layer_call = functools.partial(
    pl.kernel,
    out_type=(
        jax.ShapeDtypeStruct((NPAD, D), jnp.float32),
        jax.ShapeDtypeStruct((NPAD, D), jnp.float32),
    ),
    mesh=plsc.VectorSubcoreMesh(core_axis_name="c", subcore_axis_name="s"),
    compiler_params=pltpu.CompilerParams(needs_layout_passes=False),
    scratch_types=[
        pltpu.VMEM_SHARED((NPAD, D), jnp.float32),
        pltpu.VMEM_SHARED((NPAD, D // 2), jnp.int32),
        [pltpu.VMEM((3, C), jnp.int32) for _ in range(NPK)],
        [pltpu.VMEM((C, D), jnp.float32) for _ in range(NRW)],
        [pltpu.VMEM((C, D // 2), jnp.int32) for _ in range(NRW)],
        [pltpu.SemaphoreType.DMA for _ in range(NPK)],
        [pltpu.SemaphoreType.DMA for _ in range(NRW)],
        [pltpu.SemaphoreType.DMA for _ in range(NRW)],
    ],
)


def _readout_body(hu0, hu1, hu2, hu3, hi0, hi1, hi2, hi3,
                  users, pos, neg, u_out, p_out, n_out,
                  idx_v, ra, rb, rc, sem0, sem1, sem2):
    c = lax.axis_index("c")
    s = lax.axis_index("s")
    wid = s * NC + c

    def add_into(dst, src):
        def body(r, carry):
            for d in range(D // L):
                sl = pl.ds(d * L, L)
                dst[r, sl] = dst[r, sl] + src[r, sl]
            return carry
        lax.fori_loop(0, B_PER_W, body, 0)

    def add_scale_into(dst, src):
        def body(r, carry):
            for d in range(D // L):
                sl = pl.ds(d * L, L)
                dst[r, sl] = (dst[r, sl] + src[r, sl]) * 0.25
            return carry
        lax.fori_loop(0, B_PER_W, body, 0)

    def do_read(t0, t1, t2, t3, idx_hbm, out_hbm):
        base = wid * B_PER_W
        pltpu.sync_copy(idx_hbm.at[pl.ds(base, B_PER_W)], idx_v)
        d0 = pltpu.async_copy(t0.at[idx_v], ra, sem0)
        d1 = pltpu.async_copy(t1.at[idx_v], rb, sem1)
        d2 = pltpu.async_copy(t2.at[idx_v], rc, sem2)
        d0.wait()
        d1.wait()
        add_into(ra, rb)
        d3 = pltpu.async_copy(t3.at[idx_v], rb, sem1)
        d2.wait()
        add_into(ra, rc)
        d3.wait()
        add_scale_into(ra, rb)
        pltpu.sync_copy(ra, out_hbm.at[pl.ds(base, B_PER_W)])

    do_read(hu0, hu1, hu2, hu3, users, u_out)
    do_read(hi0, hi1, hi2, hi3, pos, p_out)
    do_read(hi0, hi1, hi2, hi3, neg, n_out)


_readout_call = functools.partial(
    pl.kernel,
    out_type=(
        jax.ShapeDtypeStruct((B, D), jnp.float32),
        jax.ShapeDtypeStruct((B, D), jnp.float32),
        jax.ShapeDtypeStruct((B, D), jnp.float32),
    ),
    mesh=plsc.VectorSubcoreMesh(core_axis_name="c", subcore_axis_name="s"),
    compiler_params=pltpu.CompilerParams(needs_layout_passes=False),
    scratch_types=[
        pltpu.VMEM((B_PER_W,), jnp.int32),
        pltpu.VMEM((B_PER_W, D), jnp.float32),
        pltpu.VMEM((B_PER_W, D), jnp.float32),
        pltpu.VMEM((B_PER_W, D), jnp.float32),
        pltpu.SemaphoreType.DMA,
        pltpu.SemaphoreType.DMA,
        pltpu.SemaphoreType.DMA,
    ],
)


def _pack_meta(gidx, sidx, norm):
    g = gidx.reshape(NCHUNK_ALL, C)
    sct = sidx.reshape(NCHUNK_ALL, C)
    nb = lax.bitcast_convert_type(norm, jnp.int32).reshape(NCHUNK_ALL, C)
    return jnp.stack([g, sct, nb], axis=1)


def kernel(user_emb, item_emb, norm_ui, norm_iu, edge_src_user,
           edge_dst_item, users, pos_items, neg_items):
    pad_e = E_PAD - E
    src = jnp.concatenate([edge_src_user, jnp.zeros((pad_e,), jnp.int32)])
    dst = jnp.concatenate([edge_dst_item, jnp.zeros((pad_e,), jnp.int32)])
    nui = jnp.concatenate([norm_ui, jnp.zeros((pad_e,), jnp.float32)])
    niu = jnp.concatenate([norm_iu, jnp.zeros((pad_e,), jnp.float32)])

    pk_ui = _pack_meta(src, dst, nui)
    pk_iu = _pack_meta(dst, src, niu)

    zrows = jnp.zeros((NPAD - N_USERS, D), jnp.float32)
    hu0 = jnp.concatenate([user_emb, zrows], axis=0)
    hi0 = jnp.concatenate([item_emb, zrows], axis=0)

    layer = _layer_call(_layer_body)
    hu1, hi1 = layer(hu0, hi0, pk_ui, pk_iu)
    hu2, hi2 = layer(hu1, hi1, pk_ui, pk_iu)
    hu3, hi3 = layer(hu2, hi2, pk_ui, pk_iu)

    u_g, p_g, n_g = _readout_call(_readout_body)(
        hu0, hu1, hu2, hu3, hi0, hi1, hi2, hi3,
        users, pos_items, neg_items)
    return (u_g, p_g, n_g)


# speedup vs baseline: 2.2672x; 2.2672x over previous
import functools

import jax
import jax.numpy as jnp
from jax import lax
from jax.experimental import pallas as pl
from jax.experimental.pallas import tpu as pltpu
from jax.experimental.pallas import tpu_sc as plsc

N_USERS = 5000
N_ITEMS = 5000
E = 320000
D = 128
B = 4096
NUM_LAYERS = 3

NC = 2
NS = 16
L = 16

NPAD = 5120
ROWS_PER_SUB = NPAD // NS
C = 128
EP = 20480
E_PAD = EP * NS
NCHUNK = EP // C
NCHUNK_ALL = NCHUNK * NS

B_PER_W = B // (NC * NS)

NPK = 4
NRW = 2
CB = 64

_BCAST_DNUMS = lax.GatherDimensionNumbers(
    offset_dims=(), collapsed_slice_dims=(0,), start_index_map=(0,))


def _bcast_lane(vec16, j):
    idx = jnp.full((L, 1), j, jnp.int32)
    return lax.gather(vec16, idx, _BCAST_DNUMS, (1,),
                      mode=lax.GatherScatterMode.PROMISE_IN_BOUNDS)


def _scale_rows(rows_ref, norm_ref):
    def body(g, carry):
        norms16 = norm_ref[pl.ds(g * L, L)]
        for j in range(L):
            e = g * L + j
            nb = _bcast_lane(norms16, j)
            for d in range(D // L):
                sl = pl.ds(d * L, L)
                rows_ref[e, sl] = rows_ref[e, sl] * nb
        return carry
    lax.fori_loop(0, C // L, body, 0)


def _layer_body(hu, hi, pk_ui, pk_iu, nm_ui, nm_iu, new_u, new_i,
                acc_sh, table_sh, pks, nms, rowss, copy_buf,
                isems, gsems, ssems):
    c = lax.axis_index("c")
    s = lax.axis_index("s")
    pk = tuple(pks)
    nm = tuple(nms)
    rows = tuple(rowss)
    isem = tuple(isems)
    gsem = tuple(gsems)
    ssem = tuple(ssems)

    z16 = jnp.zeros((L,), jnp.float32)
    def zbody(r, carry):
        for d in range(D // L):
            copy_buf[r, pl.ds(d * L, L)] = z16
        return carry
    lax.fori_loop(0, CB, zbody, 0)
    def zcp(t, carry):
        pltpu.sync_copy(copy_buf,
                        acc_sh.at[pl.ds(s * ROWS_PER_SUB + t * CB, CB)])
        return carry
    lax.fori_loop(0, ROWS_PER_SUB // CB, zcp, 0)
    plsc.subcore_barrier()

    def do_dir(table, packed, norms, out):
        cbase = s * NCHUNK

        def tcp(t, carry):
            sl = pl.ds(s * ROWS_PER_SUB + t * CB, CB)
            pltpu.sync_copy(table.at[sl], copy_buf)
            pltpu.sync_copy(copy_buf, table_sh.at[sl])
            return carry
        lax.fori_loop(0, ROWS_PER_SUB // CB, tcp, 0)
        plsc.subcore_barrier()

        def idx_start(kc, slot):
            pltpu.make_async_copy(packed.at[cbase + kc], pk[slot],
                                  isem[slot]).start()
            pltpu.make_async_copy(norms.at[cbase + kc], nm[slot],
                                  isem[slot]).start()

        def idx_wait(kc, slot):
            pltpu.make_async_copy(packed.at[cbase + kc], pk[slot],
                                  isem[slot]).wait()
            pltpu.make_async_copy(norms.at[cbase + kc], nm[slot],
                                  isem[slot]).wait()

        def gat_start(pslot, rslot):
            pltpu.make_async_copy(table_sh.at[pk[pslot].at[0]], rows[rslot],
                                  gsem[rslot]).start()

        def gat_wait(rslot):
            pltpu.make_async_copy(table_sh.at[pk[0].at[0]], rows[rslot],
                                  gsem[rslot]).wait()

        def scat_start(pslot, rslot):
            pltpu.make_async_copy(rows[rslot], acc_sh.at[pk[pslot].at[1]],
                                  ssem[rslot]).start(add=True)

        def scat_wait(rslot):
            pltpu.make_async_copy(rows[rslot], acc_sh.at[pk[0].at[1]],
                                  ssem[rslot]).wait()

        idx_start(0, 0)
        idx_start(1, 1)
        idx_wait(0, 0)
        gat_start(0, 0)

        def body4(k4, carry):
            for b in range(NPK):
                k = k4 * NPK + b
                rb = b % NRW
                rn = (b + 1) % NRW
                pn1 = (b + 1) % NPK
                pn2 = (b + 2) % NPK

                @pl.when(k < NCHUNK - 1)
                def _():
                    idx_wait(k + 1, pn1)

                if b == 0:
                    @pl.when(k > 0)
                    def _():
                        scat_wait(rn)
                else:
                    scat_wait(rn)

                @pl.when(k < NCHUNK - 1)
                def _():
                    gat_start(pn1, rn)

                @pl.when(k < NCHUNK - 2)
                def _():
                    idx_start(k + 2, pn2)

                gat_wait(rb)
                _scale_rows(rows[rb], nm[b])
                scat_start(b, rb)
            return carry

        lax.fori_loop(0, NCHUNK // NPK, body4, 0)
        scat_wait((NCHUNK - 1) % NRW)
        plsc.subcore_barrier()
        def ocp(t, carry):
            sl = pl.ds(s * ROWS_PER_SUB + t * CB, CB)
            pltpu.sync_copy(acc_sh.at[sl], copy_buf)
            pltpu.sync_copy(copy_buf, out.at[sl])
            return carry
        lax.fori_loop(0, ROWS_PER_SUB // CB, ocp, 0)

    @pl.when(c == 0)
    def _():
        do_dir(hu, pk_ui, nm_ui, new_i)

    @pl.when(c == 1)
    def _():
        do_dir(hi, pk_iu, nm_iu, new_u)


_layer_call = functools.partial(
    pl.kernel,
    out_type=(
        jax.ShapeDtypeStruct((NPAD, D), jnp.float32),
        jax.ShapeDtypeStruct((NPAD, D), jnp.float32),
    ),
    mesh=plsc.VectorSubcoreMesh(core_axis_name="c", subcore_axis_name="s"),
    scratch_types=[
        pltpu.VMEM_SHARED((NPAD, D), jnp.float32),
        pltpu.VMEM_SHARED((NPAD, D), jnp.float32),
        [pltpu.VMEM((2, C), jnp.int32) for _ in range(NPK)],
        [pltpu.VMEM((C,), jnp.float32) for _ in range(NPK)],
        [pltpu.VMEM((C, D), jnp.float32) for _ in range(NRW)],
        pltpu.VMEM((CB, D), jnp.float32),
        [pltpu.SemaphoreType.DMA for _ in range(NPK)],
        [pltpu.SemaphoreType.DMA for _ in range(NRW)],
        [pltpu.SemaphoreType.DMA for _ in range(NRW)],
    ],
)


def _readout_body(hu0, hu1, hu2, hu3, hi0, hi1, hi2, hi3,
                  users, pos, neg, u_out, p_out, n_out,
                  idx_v, ra, rb, rc, sem0, sem1, sem2):
    c = lax.axis_index("c")
    s = lax.axis_index("s")
    wid = s * NC + c

    def add_into(dst, src):
        def body(r, carry):
            for d in range(D // L):
                sl = pl.ds(d * L, L)
                dst[r, sl] = dst[r, sl] + src[r, sl]
            return carry
        lax.fori_loop(0, B_PER_W, body, 0)

    def add_scale_into(dst, src):
        def body(r, carry):
            for d in range(D // L):
                sl = pl.ds(d * L, L)
                dst[r, sl] = (dst[r, sl] + src[r, sl]) * 0.25
            return carry
        lax.fori_loop(0, B_PER_W, body, 0)

    def do_read(t0, t1, t2, t3, idx_hbm, out_hbm):
        base = wid * B_PER_W
        pltpu.sync_copy(idx_hbm.at[pl.ds(base, B_PER_W)], idx_v)
        d0 = pltpu.async_copy(t0.at[idx_v], ra, sem0)
        d1 = pltpu.async_copy(t1.at[idx_v], rb, sem1)
        d2 = pltpu.async_copy(t2.at[idx_v], rc, sem2)
        d0.wait()
        d1.wait()
        add_into(ra, rb)
        d3 = pltpu.async_copy(t3.at[idx_v], rb, sem1)
        d2.wait()
        add_into(ra, rc)
        d3.wait()
        add_scale_into(ra, rb)
        pltpu.sync_copy(ra, out_hbm.at[pl.ds(base, B_PER_W)])

    do_read(hu0, hu1, hu2, hu3, users, u_out)
    do_read(hi0, hi1, hi2, hi3, pos, p_out)
    do_read(hi0, hi1, hi2, hi3, neg, n_out)


_readout_call = functools.partial(
    pl.kernel,
    out_type=(
        jax.ShapeDtypeStruct((B, D), jnp.float32),
        jax.ShapeDtypeStruct((B, D), jnp.float32),
        jax.ShapeDtypeStruct((B, D), jnp.float32),
    ),
    mesh=plsc.VectorSubcoreMesh(core_axis_name="c", subcore_axis_name="s"),
    scratch_types=[
        pltpu.VMEM((B_PER_W,), jnp.int32),
        pltpu.VMEM((B_PER_W, D), jnp.float32),
        pltpu.VMEM((B_PER_W, D), jnp.float32),
        pltpu.VMEM((B_PER_W, D), jnp.float32),
        pltpu.SemaphoreType.DMA,
        pltpu.SemaphoreType.DMA,
        pltpu.SemaphoreType.DMA,
    ],
)


def _pack_idx(gidx, sidx):
    g = gidx.reshape(NCHUNK_ALL, C)
    sct = sidx.reshape(NCHUNK_ALL, C)
    return jnp.stack([g, sct], axis=1)


def kernel(user_emb, item_emb, norm_ui, norm_iu, edge_src_user,
           edge_dst_item, users, pos_items, neg_items):
    pad_e = E_PAD - E
    src = jnp.concatenate([edge_src_user, jnp.zeros((pad_e,), jnp.int32)])
    dst = jnp.concatenate([edge_dst_item, jnp.zeros((pad_e,), jnp.int32)])
    nui = jnp.concatenate([norm_ui, jnp.zeros((pad_e,), jnp.float32)])
    niu = jnp.concatenate([norm_iu, jnp.zeros((pad_e,), jnp.float32)])

    pk_ui = _pack_idx(src, dst)
    pk_iu = _pack_idx(dst, src)
    nm_ui = nui.reshape(NCHUNK_ALL, C)
    nm_iu = niu.reshape(NCHUNK_ALL, C)

    zrows = jnp.zeros((NPAD - N_USERS, D), jnp.float32)
    hu0 = jnp.concatenate([user_emb, zrows], axis=0)
    hi0 = jnp.concatenate([item_emb, zrows], axis=0)

    layer = _layer_call(_layer_body)
    hu1, hi1 = layer(hu0, hi0, pk_ui, pk_iu, nm_ui, nm_iu)
    hu2, hi2 = layer(hu1, hi1, pk_ui, pk_iu, nm_ui, nm_iu)
    hu3, hi3 = layer(hu2, hi2, pk_ui, pk_iu, nm_ui, nm_iu)

    u_g, p_g, n_g = _readout_call(_readout_body)(
        hu0, hu1, hu2, hu3, hi0, hi1, hi2, hi3,
        users, pos_items, neg_items)
    return (u_g, p_g, n_g)
